# Initial kernel scaffold; baseline (speedup 1.0000x reference)
import functools

import jax
import jax.numpy as jnp
from jax.experimental import pallas as pl
from jax.experimental.pallas import tpu as pltpu

B, T, D = 1, 2048, 1024
H = 1408
E = 8
BT = 256


def _silu(v):
    return v * jax.nn.sigmoid(v)


def _top2_weights(logits):
    mx = jnp.max(logits, axis=-1, keepdims=True)
    ex = jnp.exp(logits - mx)
    probs = ex / jnp.sum(ex, axis=-1, keepdims=True)
    ii = jax.lax.broadcasted_iota(jnp.int32, probs.shape, 1)
    m1 = jnp.max(probs, axis=-1, keepdims=True)
    i1 = jnp.min(jnp.where(probs == m1, ii, E), axis=-1, keepdims=True)
    mask1 = ii == i1
    p2 = jnp.where(mask1, -1.0, probs)
    m2 = jnp.max(p2, axis=-1, keepdims=True)
    i2 = jnp.min(jnp.where(p2 == m2, ii, E), axis=-1, keepdims=True)
    s = m1 + m2 + 1e-9
    return jnp.where(mask1, m1, 0.0) / s + jnp.where(ii == i2, m2, 0.0) / s


def _moe_body(x_ref, wgs_ref, wus_ref, wds_ref, wr_ref, wg_ref, wu_ref, wd_ref,
              out_ref, w_scr):
    e = pl.program_id(1)
    xb = x_ref[...]

    @pl.when(e == 0)
    def _init():
        sh = _silu(xb @ wgs_ref[...]) * (xb @ wus_ref[...])
        out_ref[...] = sh @ wds_ref[...]
        w_scr[...] = _top2_weights(xb @ wr_ref[...])

    hg = xb @ wg_ref[0]
    hu = xb @ wu_ref[0]
    y = (_silu(hg) * hu) @ wd_ref[0]
    out_ref[...] += w_scr[:, e][:, None] * y


def kernel(x, Wg_s, Wu_s, Wd_s, Wr, Wg, Wu, Wd):
    flat = x.reshape(-1, D)
    n = flat.shape[0]
    nt = n // BT
    out = pl.pallas_call(
        _moe_body,
        grid=(nt, E),
        in_specs=[
            pl.BlockSpec((BT, D), lambda t, e: (t, 0)),
            pl.BlockSpec((D, H), lambda t, e: (0, 0)),
            pl.BlockSpec((D, H), lambda t, e: (0, 0)),
            pl.BlockSpec((H, D), lambda t, e: (0, 0)),
            pl.BlockSpec((D, E), lambda t, e: (0, 0)),
            pl.BlockSpec((1, D, H), lambda t, e: (e, 0, 0)),
            pl.BlockSpec((1, D, H), lambda t, e: (e, 0, 0)),
            pl.BlockSpec((1, H, D), lambda t, e: (e, 0, 0)),
        ],
        out_specs=pl.BlockSpec((BT, D), lambda t, e: (t, 0)),
        out_shape=jax.ShapeDtypeStruct((n, D), jnp.float32),
        scratch_shapes=[pltpu.VMEM((BT, E), jnp.float32)],
        compiler_params=pltpu.CompilerParams(
            dimension_semantics=("arbitrary", "arbitrary"),
        ),
    )(flat, Wg_s, Wu_s, Wd_s, Wr, Wg, Wu, Wd)
    return out.reshape(x.shape)


# speedup vs baseline: 1.1082x; 1.1082x over previous
import functools

import jax
import jax.numpy as jnp
from jax.experimental import pallas as pl
from jax.experimental.pallas import tpu as pltpu

B, T, D = 1, 2048, 1024
H = 1408
E = 8
BT = 256


def _silu(v):
    return v * jax.nn.sigmoid(v)


def _top2_weights(logits):
    mx = jnp.max(logits, axis=-1, keepdims=True)
    ex = jnp.exp(logits - mx)
    probs = ex / jnp.sum(ex, axis=-1, keepdims=True)
    ii = jax.lax.broadcasted_iota(jnp.int32, probs.shape, 1)
    m1 = jnp.max(probs, axis=-1, keepdims=True)
    i1 = jnp.min(jnp.where(probs == m1, ii, E), axis=-1, keepdims=True)
    mask1 = ii == i1
    p2 = jnp.where(mask1, -1.0, probs)
    m2 = jnp.max(p2, axis=-1, keepdims=True)
    i2 = jnp.min(jnp.where(p2 == m2, ii, E), axis=-1, keepdims=True)
    s = m1 + m2 + 1e-9
    return jnp.where(mask1, m1, 0.0) / s + jnp.where(ii == i2, m2, 0.0) / s


def _moe_body(x_ref, wgs_ref, wus_ref, wds_ref, wr_ref, wg_ref, wu_ref, wd_ref,
              out_ref, w_scr):
    e = pl.program_id(1)
    xb = x_ref[...]

    @pl.when(e == 0)
    def _init():
        sh = _silu(xb @ wgs_ref[...]) * (xb @ wus_ref[...])
        out_ref[...] = sh @ wds_ref[...]
        w_scr[...] = _top2_weights(xb @ wr_ref[...])

    hg = xb @ wg_ref[0]
    hu = xb @ wu_ref[0]
    y = (_silu(hg) * hu) @ wd_ref[0]
    wb = w_scr[...]
    ii = jax.lax.broadcasted_iota(jnp.int32, wb.shape, 1)
    wcol = jnp.sum(jnp.where(ii == e, wb, 0.0), axis=-1, keepdims=True)
    out_ref[...] += wcol * y


def kernel(x, Wg_s, Wu_s, Wd_s, Wr, Wg, Wu, Wd):
    flat = x.reshape(-1, D)
    n = flat.shape[0]
    nt = n // BT
    out = pl.pallas_call(
        _moe_body,
        grid=(nt, E),
        in_specs=[
            pl.BlockSpec((BT, D), lambda t, e: (t, 0)),
            pl.BlockSpec((D, H), lambda t, e: (0, 0)),
            pl.BlockSpec((D, H), lambda t, e: (0, 0)),
            pl.BlockSpec((H, D), lambda t, e: (0, 0)),
            pl.BlockSpec((D, E), lambda t, e: (0, 0)),
            pl.BlockSpec((1, D, H), lambda t, e: (e, 0, 0)),
            pl.BlockSpec((1, D, H), lambda t, e: (e, 0, 0)),
            pl.BlockSpec((1, H, D), lambda t, e: (e, 0, 0)),
        ],
        out_specs=pl.BlockSpec((BT, D), lambda t, e: (t, 0)),
        out_shape=jax.ShapeDtypeStruct((n, D), jnp.float32),
        scratch_shapes=[pltpu.VMEM((BT, E), jnp.float32)],
        compiler_params=pltpu.CompilerParams(
            dimension_semantics=("arbitrary", "arbitrary"),
        ),
    )(flat, Wg_s, Wu_s, Wd_s, Wr, Wg, Wu, Wd)
    return out.reshape(x.shape)


# speedup vs baseline: 1.1121x; 1.0035x over previous
import functools

import jax
import jax.numpy as jnp
from jax.experimental import pallas as pl
from jax.experimental.pallas import tpu as pltpu

B, T, D = 1, 2048, 1024
H = 1408
E = 8
BT = 256


def _silu(v):
    return v * jax.nn.sigmoid(v)


def _mm(a, b):
    return jax.lax.dot(a.astype(jnp.bfloat16), b.astype(jnp.bfloat16),
                       preferred_element_type=jnp.float32)


def _top2_weights(logits):
    mx = jnp.max(logits, axis=-1, keepdims=True)
    ex = jnp.exp(logits - mx)
    probs = ex / jnp.sum(ex, axis=-1, keepdims=True)
    ii = jax.lax.broadcasted_iota(jnp.int32, probs.shape, 1)
    m1 = jnp.max(probs, axis=-1, keepdims=True)
    i1 = jnp.min(jnp.where(probs == m1, ii, E), axis=-1, keepdims=True)
    mask1 = ii == i1
    p2 = jnp.where(mask1, -1.0, probs)
    m2 = jnp.max(p2, axis=-1, keepdims=True)
    i2 = jnp.min(jnp.where(p2 == m2, ii, E), axis=-1, keepdims=True)
    s = m1 + m2 + 1e-9
    return jnp.where(mask1, m1, 0.0) / s + jnp.where(ii == i2, m2, 0.0) / s


def _moe_body(x_ref, wgs_ref, wus_ref, wds_ref, wr_ref, wg_ref, wu_ref, wd_ref,
              out_ref, w_scr):
    e = pl.program_id(1)
    xb = x_ref[...]

    @pl.when(e == 0)
    def _init():
        sh = _silu(_mm(xb, wgs_ref[...])) * _mm(xb, wus_ref[...])
        out_ref[...] = _mm(sh, wds_ref[...])
        w_scr[...] = _top2_weights(xb @ wr_ref[...])

    hg = _mm(xb, wg_ref[0])
    hu = _mm(xb, wu_ref[0])
    y = _mm(_silu(hg) * hu, wd_ref[0])
    wb = w_scr[...]
    ii = jax.lax.broadcasted_iota(jnp.int32, wb.shape, 1)
    wcol = jnp.sum(jnp.where(ii == e, wb, 0.0), axis=-1, keepdims=True)
    out_ref[...] += wcol * y


def kernel(x, Wg_s, Wu_s, Wd_s, Wr, Wg, Wu, Wd):
    flat = x.reshape(-1, D)
    n = flat.shape[0]
    nt = n // BT
    out = pl.pallas_call(
        _moe_body,
        grid=(nt, E),
        in_specs=[
            pl.BlockSpec((BT, D), lambda t, e: (t, 0)),
            pl.BlockSpec((D, H), lambda t, e: (0, 0)),
            pl.BlockSpec((D, H), lambda t, e: (0, 0)),
            pl.BlockSpec((H, D), lambda t, e: (0, 0)),
            pl.BlockSpec((D, E), lambda t, e: (0, 0)),
            pl.BlockSpec((1, D, H), lambda t, e: (e, 0, 0)),
            pl.BlockSpec((1, D, H), lambda t, e: (e, 0, 0)),
            pl.BlockSpec((1, H, D), lambda t, e: (e, 0, 0)),
        ],
        out_specs=pl.BlockSpec((BT, D), lambda t, e: (t, 0)),
        out_shape=jax.ShapeDtypeStruct((n, D), jnp.float32),
        scratch_shapes=[pltpu.VMEM((BT, E), jnp.float32)],
        compiler_params=pltpu.CompilerParams(
            dimension_semantics=("arbitrary", "arbitrary"),
        ),
    )(flat, Wg_s, Wu_s, Wd_s, Wr, Wg, Wu, Wd)
    return out.reshape(x.shape)


# speedup vs baseline: 1.3097x; 1.1777x over previous
import functools

import jax
import jax.numpy as jnp
from jax import lax
from jax.experimental import pallas as pl
from jax.experimental.pallas import tpu as pltpu
from jax.experimental.pallas import tpu_sc as plsc

B, T, D = 1, 2048, 1024
H = 1408
E = 8
K = 2
N = B * T
A = N * K
BA = 256
NB = A // BA + E
P = NB * BA
BT = 256

NC, NS = 2, 16
NW = NC * NS
RPW = P // NW
GCH = 64
TPW = N // NW
CT = 32


def _silu(v):
    return v * jax.nn.sigmoid(v)


def _mm(a, b):
    return jax.lax.dot(a.astype(jnp.bfloat16), b.astype(jnp.bfloat16),
                       preferred_element_type=jnp.float32)


def _router_body(x_ref, wr_ref, tp_ref, ti_ref):
    xb = x_ref[...]
    logits = xb @ wr_ref[...]
    mx = jnp.max(logits, axis=-1, keepdims=True)
    ex = jnp.exp(logits - mx)
    probs = ex / jnp.sum(ex, axis=-1, keepdims=True)
    ii = jax.lax.broadcasted_iota(jnp.int32, probs.shape, 1)
    m1 = jnp.max(probs, axis=-1, keepdims=True)
    i1 = jnp.min(jnp.where(probs == m1, ii, E), axis=-1, keepdims=True)
    p2 = jnp.where(ii == i1, -1.0, probs)
    m2 = jnp.max(p2, axis=-1, keepdims=True)
    i2 = jnp.min(jnp.where(p2 == m2, ii, E), axis=-1, keepdims=True)
    s = m1 + m2 + 1e-9
    tp_ref[...] = jnp.concatenate([m1 / s, m2 / s], axis=1)
    ti_ref[...] = jnp.concatenate([i1, i2], axis=1)


def _router(flat, Wr):
    return pl.pallas_call(
        _router_body,
        grid=(N // BT,),
        in_specs=[
            pl.BlockSpec((BT, D), lambda t: (t, 0)),
            pl.BlockSpec((D, E), lambda t: (0, 0)),
        ],
        out_specs=[
            pl.BlockSpec((BT, K), lambda t: (t, 0)),
            pl.BlockSpec((BT, K), lambda t: (t, 0)),
        ],
        out_shape=[
            jax.ShapeDtypeStruct((N, K), jnp.float32),
            jax.ShapeDtypeStruct((N, K), jnp.int32),
        ],
    )(flat, Wr)


def _shared_body(x_ref, wgs_ref, wus_ref, wds_ref, sh_ref):
    xb = x_ref[...]
    sh = _silu(_mm(xb, wgs_ref[...])) * _mm(xb, wus_ref[...])
    sh_ref[...] = _mm(sh, wds_ref[...])


def _shared(flat, Wg_s, Wu_s, Wd_s):
    return pl.pallas_call(
        _shared_body,
        grid=(N // BT,),
        in_specs=[
            pl.BlockSpec((BT, D), lambda t: (t, 0)),
            pl.BlockSpec((D, H), lambda t: (0, 0)),
            pl.BlockSpec((D, H), lambda t: (0, 0)),
            pl.BlockSpec((H, D), lambda t: (0, 0)),
        ],
        out_specs=pl.BlockSpec((BT, D), lambda t: (t, 0)),
        out_shape=jax.ShapeDtypeStruct((N, D), jnp.float32),
    )(flat, Wg_s, Wu_s, Wd_s)


def _metadata(tp, ti):
    e_a = ti.reshape(A)
    w_a = tp.reshape(A)
    toks = jnp.arange(A, dtype=jnp.int32) // K
    oh = (e_a[:, None] == jnp.arange(E, dtype=jnp.int32)).astype(jnp.int32)
    csum = jnp.cumsum(oh, axis=0)
    counts = csum[-1]
    rank = jnp.sum((csum - 1) * oh, axis=1)
    pc = (counts + BA - 1) // BA
    cum_pc = jnp.cumsum(pc)
    bstart = jnp.concatenate([jnp.zeros(1, jnp.int32), cum_pc[:-1]])
    pos = bstart[e_a] * BA + rank
    tok_arr = jnp.zeros(P, jnp.int32).at[pos].set(toks)
    wgt_arr = jnp.zeros(P, jnp.float32).at[pos].set(w_a)
    nb_used = cum_pc[-1:]
    bi = jnp.arange(NB, dtype=jnp.int32)
    blk_exp = jnp.minimum(
        jnp.sum((bi[:, None] >= cum_pc[None, :]).astype(jnp.int32), axis=1),
        E - 1)
    pos2 = pos.reshape(N, K)
    return tok_arr, wgt_arr, blk_exp, nb_used, pos2[:, 0], pos2[:, 1]


def _gather_body(tok_hbm, x_hbm, out_hbm, idx_v, rows_v, sem):
    wid = lax.axis_index("s") * NC + lax.axis_index("c")
    base = wid * RPW
    for c in range(RPW // GCH):
        off = base + c * GCH
        pltpu.sync_copy(tok_hbm.at[pl.ds(off, GCH)], idx_v)
        pltpu.async_copy(x_hbm.at[idx_v], rows_v, sem).wait()
        pltpu.sync_copy(rows_v, out_hbm.at[pl.ds(off, GCH)])


def _gather(tok_arr, flat):
    mesh = plsc.VectorSubcoreMesh(core_axis_name="c", subcore_axis_name="s")
    f = functools.partial(
        pl.kernel,
        mesh=mesh,
        out_type=jax.ShapeDtypeStruct((P, D), jnp.float32),
        scratch_types=[
            pltpu.VMEM((GCH,), jnp.int32),
            pltpu.VMEM((GCH, D), jnp.float32),
            pltpu.SemaphoreType.DMA,
        ],
    )(_gather_body)
    return f(tok_arr, flat)


def _ffn_body(be_ref, nb_ref, xs_ref, wgt_ref, wg_ref, wu_ref, wd_ref, ys_ref):
    i = pl.program_id(0)

    @pl.when(i < nb_ref[0])
    def _():
        xb = xs_ref[...]
        hg = _mm(xb, wg_ref[0])
        hu = _mm(xb, wu_ref[0])
        y = _mm(_silu(hg) * hu, wd_ref[0])
        ys_ref[...] = y * wgt_ref[0, 0, :][:, None]


def _grouped_ffn(blk_exp, nb_used, xs, wgt_arr, Wg, Wu, Wd):
    grid_spec = pltpu.PrefetchScalarGridSpec(
        num_scalar_prefetch=2,
        grid=(NB,),
        in_specs=[
            pl.BlockSpec((BA, D), lambda i, be, nb: (i, 0)),
            pl.BlockSpec((1, 1, BA), lambda i, be, nb: (i, 0, 0)),
            pl.BlockSpec((1, D, H), lambda i, be, nb: (be[i], 0, 0)),
            pl.BlockSpec((1, D, H), lambda i, be, nb: (be[i], 0, 0)),
            pl.BlockSpec((1, H, D), lambda i, be, nb: (be[i], 0, 0)),
        ],
        out_specs=pl.BlockSpec((BA, D), lambda i, be, nb: (i, 0)),
    )
    return pl.pallas_call(
        _ffn_body,
        grid_spec=grid_spec,
        out_shape=jax.ShapeDtypeStruct((P, D), jnp.float32),
        compiler_params=pltpu.CompilerParams(
            dimension_semantics=("arbitrary",),
        ),
    )(blk_exp, nb_used, xs, wgt_arr.reshape(NB, 1, BA), Wg, Wu, Wd)


def _combine_body(p0_hbm, p1_hbm, sh_hbm, ys_hbm, out_hbm,
                  i0_v, i1_v, a_v, b_v, s_v, sem):
    wid = lax.axis_index("s") * NC + lax.axis_index("c")
    base = wid * TPW
    for c in range(TPW // CT):
        tb = base + c * CT
        pltpu.sync_copy(p0_hbm.at[pl.ds(tb, CT)], i0_v)
        pltpu.sync_copy(p1_hbm.at[pl.ds(tb, CT)], i1_v)
        pltpu.async_copy(ys_hbm.at[i0_v], a_v, sem).wait()
        pltpu.async_copy(ys_hbm.at[i1_v], b_v, sem).wait()
        pltpu.sync_copy(sh_hbm.at[pl.ds(tb, CT)], s_v)

        def _row(r, _):
            def _vec(j, _):
                sl = pl.ds(j * 16, 16)
                s_v[r, sl] = s_v[r, sl] + a_v[r, sl] + b_v[r, sl]
                return 0
            return lax.fori_loop(0, D // 16, _vec, 0, unroll=4)

        lax.fori_loop(0, CT, _row, 0)
        pltpu.sync_copy(s_v, out_hbm.at[pl.ds(tb, CT)])


def _combine(p0, p1, shared_out, ys):
    mesh = plsc.VectorSubcoreMesh(core_axis_name="c", subcore_axis_name="s")
    f = functools.partial(
        pl.kernel,
        mesh=mesh,
        out_type=jax.ShapeDtypeStruct((N, D), jnp.float32),
        scratch_types=[
            pltpu.VMEM((CT,), jnp.int32),
            pltpu.VMEM((CT,), jnp.int32),
            pltpu.VMEM((CT, D), jnp.float32),
            pltpu.VMEM((CT, D), jnp.float32),
            pltpu.VMEM((CT, D), jnp.float32),
            pltpu.SemaphoreType.DMA,
        ],
    )(_combine_body)
    return f(p0, p1, shared_out, ys)


def kernel(x, Wg_s, Wu_s, Wd_s, Wr, Wg, Wu, Wd):
    flat = x.reshape(N, D)
    tp, ti = _router(flat, Wr)
    tok_arr, wgt_arr, blk_exp, nb_used, p0, p1 = _metadata(tp, ti)
    xs = _gather(tok_arr, flat)
    shared_out = _shared(flat, Wg_s, Wu_s, Wd_s)
    ys = _grouped_ffn(blk_exp, nb_used, xs, wgt_arr, Wg, Wu, Wd)
    out = _combine(p0, p1, shared_out, ys)
    return out.reshape(x.shape)
